# Initial kernel scaffold; baseline (speedup 1.0000x reference)
import jax
import jax.numpy as jnp
from jax.experimental import pallas as pl


def kernel(feature, pred, unlabeled_index, centroids):
    raise NotImplementedError("write your pallas kernel here")


# speedup vs baseline: 1.1207x; 1.1207x over previous
import jax
import jax.numpy as jnp
from jax import lax
from jax.experimental import pallas as pl

_B = 16384
_C = 1000
_CP = 1024
_F = 16
_BM = 512


def _cluster_body(x_ref, ct_ref, out_ref):
    x = x_ref[...]
    ct = ct_ref[...]
    lane = lax.broadcasted_iota(jnp.int32, (1, _CP), 1)
    pad = lane >= _C

    xx = jnp.sum(x * x, axis=1, keepdims=True)
    yy = jnp.sum(ct * ct, axis=0, keepdims=True)

    g = lax.dot_general(x, ct, (((1,), (0,)), ((), ())),
                        preferred_element_type=jnp.float32)
    d = (xx + (yy + jnp.where(pad, 3e38, 0.0))) - 2.0 * g
    dmin = jnp.min(d, axis=1, keepdims=True)
    iota = lax.broadcasted_iota(jnp.int32, (_BM, _CP), 1)
    e_id = jnp.min(jnp.where(d == dmin, iota, _CP), axis=1, keepdims=True)

    fn = x / jnp.clip(jnp.sqrt(xx), 1e-8, None)
    cnt = ct / jnp.clip(jnp.sqrt(yy), 1e-8, None)
    cos = lax.dot_general(fn, cnt, (((1,), (0,)), ((), ())),
                          preferred_element_type=jnp.float32)
    cosm = cos + jnp.where(pad, -3e38, 0.0)
    cmax = jnp.max(cosm, axis=1, keepdims=True)
    c_id = jnp.min(jnp.where(cosm == cmax, iota, _CP), axis=1, keepdims=True)

    accept = (c_id == e_id) & (cmax > 0.85)
    out_ref[...] = jnp.where(accept, c_id.astype(jnp.float32), -1.0)


def kernel(feature, pred, unlabeled_index, centroids):
    del pred, unlabeled_index
    ct = jnp.pad(centroids, ((0, _CP - _C), (0, 0))).T
    out = pl.pallas_call(
        _cluster_body,
        grid=(_B // _BM,),
        in_specs=[
            pl.BlockSpec((_BM, _F), lambda i: (i, 0)),
            pl.BlockSpec((_F, _CP), lambda i: (0, 0)),
        ],
        out_specs=pl.BlockSpec((_BM, 1), lambda i: (i, 0)),
        out_shape=jax.ShapeDtypeStruct((_B, 1), jnp.float32),
    )(feature, ct)
    return out[:, 0]


# speedup vs baseline: 1.3030x; 1.1626x over previous
import jax
import jax.numpy as jnp
from jax import lax
from jax.experimental import pallas as pl

_B = 16384
_C = 1000
_CP = 1024
_F = 16
_BM = 1024


def _cluster_body(x_ref, ct_ref, iota_ref, out_ref):
    x = x_ref[...]
    ct = ct_ref[...]
    iota = iota_ref[...]
    lane = lax.broadcasted_iota(jnp.int32, (1, _CP), 1)
    pad = lane >= _C

    xx = jnp.sum(x * x, axis=1, keepdims=True)
    yy = jnp.sum(ct * ct, axis=0, keepdims=True)

    g2 = lax.dot_general(x, ct + ct, (((1,), (0,)), ((), ())),
                         preferred_element_type=jnp.float32)
    d = (xx + (yy + jnp.where(pad, 3e38, 0.0))) - g2
    dmin = jnp.min(d, axis=1, keepdims=True)
    e_id = jnp.min(jnp.where(d == dmin, iota, float(_CP)), axis=1, keepdims=True)

    fn = x / jnp.clip(jnp.sqrt(xx), 1e-8, None)
    cnt = ct / jnp.clip(jnp.sqrt(yy), 1e-8, None)
    cos = lax.dot_general(fn, cnt, (((1,), (0,)), ((), ())),
                          preferred_element_type=jnp.float32)
    cmax = jnp.max(cos, axis=1, keepdims=True)
    c_id = jnp.min(jnp.where(cos == cmax, iota, float(_CP)), axis=1, keepdims=True)

    accept = (c_id == e_id) & (cmax > 0.85)
    out_ref[...] = jnp.where(accept, c_id, -1.0)


def kernel(feature, pred, unlabeled_index, centroids):
    del pred, unlabeled_index
    ct = jnp.pad(centroids, ((0, _CP - _C), (0, 0))).T
    iota = jnp.arange(_CP, dtype=jnp.float32)[None, :]
    out = pl.pallas_call(
        _cluster_body,
        grid=(_B // _BM,),
        in_specs=[
            pl.BlockSpec((_BM, _F), lambda i: (i, 0)),
            pl.BlockSpec((_F, _CP), lambda i: (0, 0)),
            pl.BlockSpec((1, _CP), lambda i: (0, 0)),
        ],
        out_specs=pl.BlockSpec((_BM, 1), lambda i: (i, 0)),
        out_shape=jax.ShapeDtypeStruct((_B, 1), jnp.float32),
    )(feature, ct, iota)
    return out[:, 0]


# speedup vs baseline: 1.6120x; 1.2371x over previous
import jax
import jax.numpy as jnp
from jax import lax
from jax.experimental import pallas as pl

_B = 16384
_C = 1000
_F = 16
_BM = 1024


def _cluster_body(c_ref, xt_ref, iota_ref, out_ref):
    c = c_ref[...]
    xt = xt_ref[...]
    iota = iota_ref[...]

    xx = jnp.sum(xt * xt, axis=0, keepdims=True)
    yy = jnp.sum(c * c, axis=1, keepdims=True)

    g2 = lax.dot_general(c + c, xt, (((1,), (0,)), ((), ())),
                         preferred_element_type=jnp.float32)
    d = (xx + yy) - g2
    dmin = jnp.min(d, axis=0, keepdims=True)
    e_id = jnp.min(jnp.where(d == dmin, iota, float(_C)), axis=0, keepdims=True)

    fn = xt / jnp.clip(jnp.sqrt(xx), 1e-8, None)
    cn = c / jnp.clip(jnp.sqrt(yy), 1e-8, None)
    cos = lax.dot_general(cn, fn, (((1,), (0,)), ((), ())),
                          preferred_element_type=jnp.float32)
    cmax = jnp.max(cos, axis=0, keepdims=True)
    c_id = jnp.min(jnp.where(cos == cmax, iota, float(_C)), axis=0, keepdims=True)

    accept = (c_id == e_id) & (cmax > 0.85)
    out_ref[...] = jnp.where(accept, c_id, -1.0)


def kernel(feature, pred, unlabeled_index, centroids):
    del pred, unlabeled_index
    xt = feature.T
    iota = jnp.arange(_C, dtype=jnp.float32)[:, None]
    out = pl.pallas_call(
        _cluster_body,
        grid=(_B // _BM,),
        in_specs=[
            pl.BlockSpec((_C, _F), lambda i: (0, 0)),
            pl.BlockSpec((_F, _BM), lambda i: (0, i)),
            pl.BlockSpec((_C, 1), lambda i: (0, 0)),
        ],
        out_specs=pl.BlockSpec((1, _BM), lambda i: (0, i)),
        out_shape=jax.ShapeDtypeStruct((1, _B), jnp.float32),
    )(centroids, xt, iota)
    return out[0]


# speedup vs baseline: 2.2040x; 1.3673x over previous
import jax
import jax.numpy as jnp
from jax import lax
from jax.experimental import pallas as pl

_B = 16384
_C = 1000
_F = 16
_BM = 1024


def _cluster_body(c_ref, xt_ref, out_ref):
    c = c_ref[...]
    xt = xt_ref[...]

    xx = jnp.sum(xt * xt, axis=0, keepdims=True)
    yy = jnp.sum(c * c, axis=1, keepdims=True)

    g2 = lax.dot_general(c + c, xt, (((1,), (0,)), ((), ())),
                         preferred_element_type=jnp.float32)
    d = (xx + yy) - g2
    e_id = jnp.argmin(d, axis=0)[None, :].astype(jnp.float32)

    fn = xt / jnp.clip(jnp.sqrt(xx), 1e-8, None)
    cn = c / jnp.clip(jnp.sqrt(yy), 1e-8, None)
    cos = lax.dot_general(cn, fn, (((1,), (0,)), ((), ())),
                          preferred_element_type=jnp.float32)
    cmax = jnp.max(cos, axis=0, keepdims=True)
    c_id = jnp.argmax(cos, axis=0)[None, :].astype(jnp.float32)

    accept = (c_id == e_id) & (cmax > 0.85)
    out_ref[...] = jnp.where(accept, c_id, -1.0)


def kernel(feature, pred, unlabeled_index, centroids):
    del pred, unlabeled_index
    xt = feature.T
    out = pl.pallas_call(
        _cluster_body,
        grid=(_B // _BM,),
        in_specs=[
            pl.BlockSpec((_C, _F), lambda i: (0, 0)),
            pl.BlockSpec((_F, _BM), lambda i: (0, i)),
        ],
        out_specs=pl.BlockSpec((1, _BM), lambda i: (0, i)),
        out_shape=jax.ShapeDtypeStruct((1, _B), jnp.float32),
    )(centroids, xt)
    return out[0]


# speedup vs baseline: 2.4578x; 1.1152x over previous
import jax
import jax.numpy as jnp
from jax import lax
from jax.experimental import pallas as pl

_B = 16384
_C = 1000
_F = 16
_BM = 2048


def _cluster_body(c_ref, xt_ref, out_ref):
    c = c_ref[...]
    xt = xt_ref[...]

    xx = jnp.sum(xt * xt, axis=0, keepdims=True)
    yy = jnp.sum(c * c, axis=1, keepdims=True)

    g2 = lax.dot_general(c + c, xt, (((1,), (0,)), ((), ())),
                         preferred_element_type=jnp.float32)
    d = (xx + yy) - g2
    e_id = jnp.argmin(d, axis=0)[None, :].astype(jnp.float32)

    fn = xt / jnp.clip(jnp.sqrt(xx), 1e-8, None)
    cn = c / jnp.clip(jnp.sqrt(yy), 1e-8, None)
    cos = lax.dot_general(cn, fn, (((1,), (0,)), ((), ())),
                          preferred_element_type=jnp.float32)
    cmax = jnp.max(cos, axis=0, keepdims=True)
    c_id = jnp.argmax(cos, axis=0)[None, :].astype(jnp.float32)

    accept = (c_id == e_id) & (cmax > 0.85)
    out_ref[...] = jnp.where(accept, c_id, -1.0)


def kernel(feature, pred, unlabeled_index, centroids):
    del pred, unlabeled_index
    xt = feature.T
    out = pl.pallas_call(
        _cluster_body,
        grid=(_B // _BM,),
        in_specs=[
            pl.BlockSpec((_C, _F), lambda i: (0, 0)),
            pl.BlockSpec((_F, _BM), lambda i: (0, i)),
        ],
        out_specs=pl.BlockSpec((1, _BM), lambda i: (0, i)),
        out_shape=jax.ShapeDtypeStruct((1, _B), jnp.float32),
    )(centroids, xt)
    return out[0]


# speedup vs baseline: 2.5583x; 1.0409x over previous
import jax
import jax.numpy as jnp
from jax import lax
from jax.experimental import pallas as pl

_B = 16384
_C = 1000
_F = 16
_BM = 4096


def _cluster_body(c_ref, xt_ref, out_ref):
    c = c_ref[...]
    xt = xt_ref[...]

    xx = jnp.sum(xt * xt, axis=0, keepdims=True)
    yy = jnp.sum(c * c, axis=1, keepdims=True)

    g2 = lax.dot_general(c + c, xt, (((1,), (0,)), ((), ())),
                         preferred_element_type=jnp.float32)
    d = (xx + yy) - g2
    e_id = jnp.argmin(d, axis=0)[None, :].astype(jnp.float32)

    fn = xt / jnp.clip(jnp.sqrt(xx), 1e-8, None)
    cn = c / jnp.clip(jnp.sqrt(yy), 1e-8, None)
    cos = lax.dot_general(cn, fn, (((1,), (0,)), ((), ())),
                          preferred_element_type=jnp.float32)
    cmax = jnp.max(cos, axis=0, keepdims=True)
    c_id = jnp.argmax(cos, axis=0)[None, :].astype(jnp.float32)

    accept = (c_id == e_id) & (cmax > 0.85)
    out_ref[...] = jnp.where(accept, c_id, -1.0)


def kernel(feature, pred, unlabeled_index, centroids):
    del pred, unlabeled_index
    xt = feature.T
    out = pl.pallas_call(
        _cluster_body,
        grid=(_B // _BM,),
        in_specs=[
            pl.BlockSpec((_C, _F), lambda i: (0, 0)),
            pl.BlockSpec((_F, _BM), lambda i: (0, i)),
        ],
        out_specs=pl.BlockSpec((1, _BM), lambda i: (0, i)),
        out_shape=jax.ShapeDtypeStruct((1, _B), jnp.float32),
    )(centroids, xt)
    return out[0]
